# baseline (speedup 1.0000x reference)
import functools

import jax
import jax.numpy as jnp
from jax import lax
from jax.experimental import pallas as pl
from jax.experimental.pallas import tpu as pltpu
from jax.experimental.pallas import tpu_sc as plsc

PRE_SEQ_LEN = 128
HIDDEN = 1024
OUT_DIM = 24 * 2 * 1024
TOKENS = 8 * 128
CW = 1024
NCHUNK = OUT_DIM // CW

NC, NS = 2, 16
NW = NC * NS
TPW = TOKENS // NW
ROWS_PER_TEC = PRE_SEQ_LEN // NS


def _table_body(emb_ref, w1_ref, b1_ref, w2_ref, b2_ref, tab_ref, h_ref):
    j = pl.program_id(0)

    @pl.when(j == 0)
    def _init():
        h_ref[...] = jnp.tanh(
            jnp.dot(emb_ref[...], w1_ref[...],
                    preferred_element_type=jnp.float32) + b1_ref[...])

    t = jnp.dot(h_ref[...], w2_ref[...],
                preferred_element_type=jnp.float32) + b2_ref[...]
    tab_ref[...] = t.reshape(1, PRE_SEQ_LEN, CW)


def _make_table(emb, W1, b1, W2, b2):
    return pl.pallas_call(
        _table_body,
        grid=(NCHUNK,),
        in_specs=[
            pl.BlockSpec((PRE_SEQ_LEN, HIDDEN), lambda j: (0, 0)),
            pl.BlockSpec((HIDDEN, HIDDEN), lambda j: (0, 0)),
            pl.BlockSpec((1, HIDDEN), lambda j: (0, 0)),
            pl.BlockSpec((HIDDEN, CW), lambda j: (0, j)),
            pl.BlockSpec((1, CW), lambda j: (0, j)),
        ],
        out_specs=pl.BlockSpec((1, PRE_SEQ_LEN, CW), lambda j: (j, 0, 0)),
        out_shape=jax.ShapeDtypeStruct((NCHUNK, PRE_SEQ_LEN, CW), jnp.float32),
        scratch_shapes=[pltpu.VMEM((PRE_SEQ_LEN, HIDDEN), jnp.float32)],
        compiler_params=pltpu.CompilerParams(
            dimension_semantics=("arbitrary",),
        ),
    )(emb, W1, b1.reshape(1, HIDDEN), W2, b2.reshape(1, OUT_DIM))


@functools.partial(
    pl.kernel,
    out_type=jax.ShapeDtypeStruct((TOKENS, OUT_DIM), jnp.float32),
    mesh=plsc.VectorSubcoreMesh(core_axis_name="c", subcore_axis_name="s"),
    scratch_types=[
        pltpu.VMEM((TPW,), jnp.int32),
        pltpu.VMEM((TPW, CW), jnp.float32),
        pltpu.VMEM_SHARED((PRE_SEQ_LEN, CW), jnp.float32),
        pltpu.SemaphoreType.DMA,
    ],
)
def _sc_gather(tab_hbm, idx_hbm, out_hbm, idx_v, buf_v, chunk_s, sem):
    cid = lax.axis_index("c")
    sid = lax.axis_index("s")
    wid = sid * NC + cid
    base = wid * TPW
    pltpu.sync_copy(idx_hbm.at[pl.ds(base, TPW)], idx_v)

    @pl.loop(0, NCHUNK)
    def _chunk(c):
        pltpu.sync_copy(tab_hbm.at[c, pl.ds(sid * ROWS_PER_TEC, ROWS_PER_TEC)],
                        chunk_s.at[pl.ds(sid * ROWS_PER_TEC, ROWS_PER_TEC)])
        plsc.subcore_barrier()
        idx_lo = idx_v[pl.ds(0, 16)]
        idx_hi = idx_v[pl.ds(16, 16)]
        for i in range(TPW):
            r = idx_lo[i] if i < 16 else idx_hi[i - 16]
            pltpu.async_copy(chunk_s.at[r], buf_v.at[i], sem)
        pltpu.make_async_copy(tab_hbm.at[0].at[pl.ds(0, TPW)], buf_v, sem).wait()
        pltpu.sync_copy(buf_v, out_hbm.at[pl.ds(base, TPW), pl.ds(c * CW, CW)])
        plsc.subcore_barrier()


def kernel(prefix, emb, W1, b1, W2, b2):
    table = _make_table(emb, W1, b1, W2, b2)
    idx = prefix.reshape(TOKENS).astype(jnp.int32)
    out = _sc_gather(table, idx)
    return out.reshape(prefix.shape[0], prefix.shape[1], OUT_DIM)


# speedup vs baseline: 2.2302x; 2.2302x over previous
import jax
import jax.numpy as jnp
from jax.experimental import pallas as pl
from jax.experimental.pallas import tpu as pltpu

PRE_SEQ_LEN = 128
HIDDEN = 1024
OUT_DIM = 24 * 2 * 1024
TOKENS = 8 * 128
BLK = 2048


def _body(prefix_ref, emb_ref, w1_ref, b1_ref, w2_ref, b2_ref, out_ref,
          h_ref, oh_ref):
    j = pl.program_id(0)

    @pl.when(j == 0)
    def _init():
        h_ref[...] = jnp.tanh(
            jnp.dot(emb_ref[...], w1_ref[...],
                    preferred_element_type=jnp.float32) + b1_ref[...]
        ).astype(jnp.bfloat16)
        row_ids = jax.lax.broadcasted_iota(jnp.int32, (TOKENS, PRE_SEQ_LEN), 1)
        oh_ref[...] = (prefix_ref[...] == row_ids).astype(jnp.bfloat16)

    t = jnp.dot(h_ref[...], w2_ref[...].astype(jnp.bfloat16),
                preferred_element_type=jnp.float32)
    out_ref[...] = jnp.dot(oh_ref[...], t.astype(jnp.bfloat16),
                           preferred_element_type=jnp.float32) + b2_ref[...]


def kernel(prefix, emb, W1, b1, W2, b2):
    prefix2d = prefix.reshape(TOKENS, 1).astype(jnp.int32)
    b1r = b1.reshape(1, HIDDEN)
    b2r = b2.reshape(1, OUT_DIM)
    grid = (OUT_DIM // BLK,)
    out = pl.pallas_call(
        _body,
        grid=grid,
        in_specs=[
            pl.BlockSpec((TOKENS, 1), lambda j: (0, 0)),
            pl.BlockSpec((PRE_SEQ_LEN, HIDDEN), lambda j: (0, 0)),
            pl.BlockSpec((HIDDEN, HIDDEN), lambda j: (0, 0)),
            pl.BlockSpec((1, HIDDEN), lambda j: (0, 0)),
            pl.BlockSpec((HIDDEN, BLK), lambda j: (0, j)),
            pl.BlockSpec((1, BLK), lambda j: (0, j)),
        ],
        out_specs=pl.BlockSpec((TOKENS, BLK), lambda j: (0, j)),
        out_shape=jax.ShapeDtypeStruct((TOKENS, OUT_DIM), jnp.float32),
        scratch_shapes=[
            pltpu.VMEM((PRE_SEQ_LEN, HIDDEN), jnp.bfloat16),
            pltpu.VMEM((TOKENS, PRE_SEQ_LEN), jnp.bfloat16),
        ],
        compiler_params=pltpu.CompilerParams(
            dimension_semantics=("arbitrary",),
        ),
    )(prefix2d, emb, W1, b1r, W2, b2r)
    return out.reshape(prefix.shape[0], prefix.shape[1], OUT_DIM)


# speedup vs baseline: 2.2766x; 1.0208x over previous
import jax
import jax.numpy as jnp
from jax.experimental import pallas as pl
from jax.experimental.pallas import tpu as pltpu

PRE_SEQ_LEN = 128
HIDDEN = 1024
OUT_DIM = 24 * 2 * 1024
TOKENS = 8 * 128
BLK = 2048
NBLK = OUT_DIM // BLK


def _body(prefix_ref, emb_ref, w1_ref, b1_ref, w2_ref, b2_ref, out_ref,
          h_ref, oh_ref, t_ref):
    p = pl.program_id(0)
    j = pl.program_id(1)

    @pl.when((p == 0) & (j == 0))
    def _init():
        h_ref[...] = jnp.tanh(
            jnp.dot(emb_ref[...], w1_ref[...],
                    preferred_element_type=jnp.float32) + b1_ref[...]
        ).astype(jnp.bfloat16)
        row_ids = jax.lax.broadcasted_iota(jnp.int32, (TOKENS, PRE_SEQ_LEN), 1)
        oh_ref[...] = (prefix_ref[...] == row_ids).astype(jnp.bfloat16)

    @pl.when(p == 0)
    def _compute_table():
        t = jnp.dot(h_ref[...], w2_ref[...].astype(jnp.bfloat16),
                    preferred_element_type=jnp.float32)
        t_ref[:, pl.ds(j * BLK, BLK)] = t.astype(jnp.bfloat16)

    @pl.when(p == 1)
    def _expand():
        out_ref[...] = jnp.dot(oh_ref[...], t_ref[:, pl.ds(j * BLK, BLK)],
                               preferred_element_type=jnp.float32) + b2_ref[...]


def kernel(prefix, emb, W1, b1, W2, b2):
    prefix2d = prefix.reshape(TOKENS, 1).astype(jnp.int32)
    b1r = b1.reshape(1, HIDDEN)
    b2r = b2.reshape(1, OUT_DIM)
    out = pl.pallas_call(
        _body,
        grid=(2, NBLK),
        in_specs=[
            pl.BlockSpec((TOKENS, 1), lambda p, j: (0, 0)),
            pl.BlockSpec((PRE_SEQ_LEN, HIDDEN), lambda p, j: (0, 0)),
            pl.BlockSpec((HIDDEN, HIDDEN), lambda p, j: (0, 0)),
            pl.BlockSpec((1, HIDDEN), lambda p, j: (0, 0)),
            pl.BlockSpec((HIDDEN, BLK),
                         lambda p, j: (0, jnp.where(p == 0, j, NBLK - 1))),
            pl.BlockSpec((1, BLK), lambda p, j: (0, j)),
        ],
        out_specs=pl.BlockSpec((TOKENS, BLK),
                               lambda p, j: (0, jnp.where(p == 0, 0, j))),
        out_shape=jax.ShapeDtypeStruct((TOKENS, OUT_DIM), jnp.float32),
        scratch_shapes=[
            pltpu.VMEM((PRE_SEQ_LEN, HIDDEN), jnp.bfloat16),
            pltpu.VMEM((TOKENS, PRE_SEQ_LEN), jnp.bfloat16),
            pltpu.VMEM((PRE_SEQ_LEN, OUT_DIM), jnp.bfloat16),
        ],
        compiler_params=pltpu.CompilerParams(
            dimension_semantics=("arbitrary", "arbitrary"),
        ),
    )(prefix2d, emb, W1, b1r, W2, b2r)
    return out.reshape(prefix.shape[0], prefix.shape[1], OUT_DIM)
